# baseline (speedup 1.0000x reference)
import functools

import jax
import jax.numpy as jnp
from jax import lax
from jax.experimental import pallas as pl
from jax.experimental.pallas import tpu as pltpu
from jax.experimental.pallas import tpu_sc as plsc

_NC = 2
_NS = 16
_NW = _NC * _NS
_L = 16


def _sc_body(CH, B, N, C, k, pts_hbm, idx_hbm, out_hbm,
             idxbs, rids, rowss, stags, isems, gsems, osems):
    PPT = N * B // _NW
    KC = k * CH
    NR = KC + _L
    n_chunks = PPT // CH
    NI = len(idxbs)

    wid = lax.axis_index("s") * _NC + lax.axis_index("c")
    base_pt = wid * PPT
    b = base_pt // N
    boff2 = b * (N // 2)
    n_base = base_pt - b * N

    io = lax.iota(jnp.int32, _L)
    ccol = [io + cc * _L for cc in range(C // _L)]
    ccol2 = [c_ + C for c_ in ccol]

    def idx_copy(ch, q):
        return pltpu.make_async_copy(
            idx_hbm.at[pl.ds(base_pt + ch * CH, CH)], idxbs[q], isems[q])

    def gather_copy(q):
        return pltpu.make_async_copy(pts_hbm.at[rids[q]], rowss[q], gsems[q])

    def out_copy(ch, q):
        return pltpu.make_async_copy(
            stags[q], out_hbm.at[b, pl.ds(n_base + ch * CH, CH)], osems[q])

    def prep_rid(iq, gq):
        for p in range(CH):
            v_a = (idxbs[iq][p, pl.ds(0, _L)] >> 1) + boff2
            v_b = (idxbs[iq][p, pl.ds(k - _L, _L)] >> 1) + boff2
            rids[gq][pl.ds(p * k, _L)] = v_a
            rids[gq][pl.ds(p * k + (k - _L), _L)] = v_b

    def prep_tail(ch, gq):
        p0h = boff2 + (n_base + ch * CH) // 2
        rids[gq][pl.ds(KC, _L)] = lax.broadcast(p0h, (_L,))

    def compute(iq, gq):
        rows = rowss[gq]
        stag = stags[gq]
        for p in range(CH):
            xr = [rows[KC, pl.ds(p * C + cc * _L, _L)]
                  for cc in range(C // _L)]
            pvec = lax.broadcast(p, (_L,))
            pv_a = idxbs[iq][p, pl.ds(0, _L)] & 1
            pv_b = idxbs[iq][p, pl.ds(k - _L, _L)] & 1
            for j in range(k):
                par = pv_a[j] if j < _L else pv_b[j - (k - _L)]
                jvec = lax.broadcast(j, (_L,))
                for cc in range(C // _L):
                    g = rows[p * k + j, pl.ds(par * C + cc * _L, _L)]
                    plsc.store_scatter(stag, [pvec, ccol[cc], jvec],
                                       g - xr[cc])
                    plsc.store_scatter(stag, [pvec, ccol2[cc], jvec],
                                       xr[cc])

    idx_copy(0, 0).start()
    idx_copy(1, 1).start()
    idx_copy(0, 0).wait()
    prep_rid(0, 0)
    prep_tail(0, 0)
    gather_copy(0).start()

    RING = NI

    def ring_body(i, _):
        ch0 = i * RING
        for s in range(RING):
            ch = ch0 + s
            iq = s
            gq = s % 2
            iqn = (s + 1) % NI
            gqn = (s + 1) % 2
            @pl.when(ch + 2 < n_chunks)
            def _():
                idx_copy(ch + 2, (s + 2) % NI).start()
            @pl.when(ch + 1 < n_chunks)
            def _():
                idx_copy(ch + 1, iqn).wait()
                prep_rid(iqn, gqn)
                prep_tail(ch + 1, gqn)
                gather_copy(gqn).start()
            gather_copy(gq).wait()
            if s < 2:
                @pl.when(i > 0)
                def _():
                    out_copy(ch, gq).wait()
            else:
                out_copy(ch, gq).wait()
            compute(iq, gq)
            out_copy(ch, gq).start()
        return ()

    lax.fori_loop(0, n_chunks // RING, ring_body, (), unroll=False)
    out_copy(n_chunks - 2, 0).wait()
    out_copy(n_chunks - 1, 1).wait()


def kernel(points, idx):
    B, N, C = points.shape
    k = idx.shape[2]
    BN = B * N
    CH = 2

    pts2 = points.reshape(BN // 2, 2 * C)
    idx2 = idx.reshape(BN, k)

    mesh = plsc.VectorSubcoreMesh(core_axis_name="c", subcore_axis_name="s")
    body = functools.partial(_sc_body, CH, B, N, C, k)
    NR = k * CH + _L
    sc_fn = pl.kernel(
        body,
        out_type=jax.ShapeDtypeStruct((B, N, 2 * C, k), jnp.float32),
        mesh=mesh,
        compiler_params=pltpu.CompilerParams(needs_layout_passes=False,
                                             use_tc_tiling_on_sc=True),
        scratch_types=[
            [pltpu.VMEM((CH, k), jnp.int32) for _ in range(4)],
            [pltpu.VMEM((NR,), jnp.int32) for _ in range(2)],
            [pltpu.VMEM((NR, 2 * C), jnp.float32) for _ in range(2)],
            [pltpu.VMEM((CH, 2 * C, k), jnp.float32) for _ in range(2)],
            [pltpu.SemaphoreType.DMA for _ in range(4)],
            [pltpu.SemaphoreType.DMA for _ in range(2)],
            [pltpu.SemaphoreType.DMA for _ in range(2)],
        ],
    )
    return sc_fn(pts2, idx2)


# speedup vs baseline: 4.1352x; 4.1352x over previous
import functools

import jax
import jax.numpy as jnp
from jax import lax
from jax.experimental import pallas as pl
from jax.experimental.pallas import tpu as pltpu
from jax.experimental.pallas import tpu_sc as plsc

_NC = 2
_NS = 16
_NW = _NC * _NS
_L = 16


def _sc_body(CH, B, N, C, k, pts_hbm, idx_hbm, out_hbm,
             idx_v, rids, rowss, stags, gsems, osems):
    PPT = N * B // _NW
    KC = k * CH
    NR = KC + _L
    n_chunks = PPT // CH

    wid = lax.axis_index("s") * _NC + lax.axis_index("c")
    base_pt = wid * PPT
    b = base_pt // N
    boff2 = b * (N // 2)
    n_base = base_pt - b * N

    io = lax.iota(jnp.int32, _L)

    pltpu.sync_copy(idx_hbm.at[wid], idx_v)

    def gather_copy(q):
        return pltpu.make_async_copy(pts_hbm.at[rids[q]], rowss[q], gsems[q])

    def out_copy(ch, q):
        return pltpu.make_async_copy(
            stags[q],
            out_hbm.at[b, :, pl.ds(n_base + ch * CH, CH)],
            osems[q])

    def prep_rid(ch, q):
        q0 = ch * KC
        for p in range(CH):
            v_a = (idx_v[pl.ds(q0 + p * k, _L)] >> 1) + boff2
            v_b = (idx_v[pl.ds(q0 + p * k + (k - _L), _L)] >> 1) + boff2
            rids[q][pl.ds(p * k, _L)] = v_a
            rids[q][pl.ds(p * k + (k - _L), _L)] = v_b
        p0h = boff2 + (n_base + ch * CH) // 2
        tail = jnp.where(io < CH // 2, io + p0h, p0h)
        rids[q][pl.ds(KC, _L)] = tail

    def compute(ch, q):
        q0 = ch * KC
        rows = rowss[q]
        stag = stags[q]
        for p in range(CH):
            xr = [rows[KC + p // 2, pl.ds((p % 2) * C + cc * _L, _L)]
                  for cc in range(C // _L)]
            pv_a = idx_v[pl.ds(q0 + p * k, _L)] & 1
            pv_b = idx_v[pl.ds(q0 + p * k + (k - _L), _L)] & 1
            for j in range(k):
                par = pv_a[j] if j < _L else pv_b[j - (k - _L)]
                off = par * C
                for cc in range(C // _L):
                    g = rows[p * k + j, pl.ds(off + cc * _L, _L)]
                    stag[j, p, pl.ds(cc * _L, _L)] = g - xr[cc]
                    stag[j, p, pl.ds(C + cc * _L, _L)] = xr[cc]

    prep_rid(0, 0)
    gather_copy(0).start()

    def pair_body(i, _):
        for s in range(2):
            ch = i * 2 + s
            @pl.when(ch + 1 < n_chunks)
            def _():
                prep_rid(ch + 1, 1 - s)
                gather_copy(1 - s).start()
            gather_copy(s).wait()
            @pl.when(i > 0)
            def _():
                out_copy(ch, s).wait()
            compute(ch, s)
            out_copy(ch, s).start()
        return ()

    lax.fori_loop(0, n_chunks // 2, pair_body, (), unroll=False)
    out_copy(n_chunks - 2, 0).wait()
    out_copy(n_chunks - 1, 1).wait()


def kernel(points, idx):
    B, N, C = points.shape
    k = idx.shape[2]
    BN = B * N
    CH = 8

    pts2 = points.reshape(BN // 2, 2 * C)
    idx_t = idx.reshape(_NW, (BN // _NW) * k)

    mesh = plsc.VectorSubcoreMesh(core_axis_name="c", subcore_axis_name="s")
    body = functools.partial(_sc_body, CH, B, N, C, k)
    NR = k * CH + _L
    sc_fn = pl.kernel(
        body,
        out_type=jax.ShapeDtypeStruct((B, k, N, 2 * C), jnp.float32),
        mesh=mesh,
        compiler_params=pltpu.CompilerParams(needs_layout_passes=False,
                                             use_tc_tiling_on_sc=True),
        scratch_types=[
            pltpu.VMEM(((BN // _NW) * k,), jnp.int32),
            [pltpu.VMEM((NR,), jnp.int32) for _ in range(2)],
            [pltpu.VMEM((NR, 2 * C), jnp.float32) for _ in range(2)],
            [pltpu.VMEM((k, CH, 2 * C), jnp.float32) for _ in range(2)],
            [pltpu.SemaphoreType.DMA for _ in range(2)],
            [pltpu.SemaphoreType.DMA for _ in range(2)],
        ],
    )
    out = sc_fn(pts2, idx_t)
    return jnp.transpose(out, (0, 2, 3, 1))


# speedup vs baseline: 6.9475x; 1.6801x over previous
import functools

import jax
import jax.numpy as jnp
from jax import lax
from jax.experimental import pallas as pl
from jax.experimental.pallas import tpu as pltpu
from jax.experimental.pallas import tpu_sc as plsc

_NC = 2
_NS = 16
_NW = _NC * _NS
_L = 16


def _sc_body(CH, B, N, C, k, pts_hbm, idx_hbm, out_hbm,
             idx_v, rids, rowss, stags, gsems, osems):
    PPT = N * B // _NW
    KC = k * CH
    NR = KC + _L
    n_chunks = PPT // CH

    wid = lax.axis_index("s") * _NC + lax.axis_index("c")
    base_pt = wid * PPT
    b = base_pt // N
    boff = b * N
    n_base = base_pt - b * N

    io = lax.iota(jnp.int32, _L)

    pltpu.sync_copy(idx_hbm.at[wid], idx_v)

    def gather_copy(q):
        return pltpu.make_async_copy(pts_hbm.at[rids[q]], rowss[q], gsems[q])

    def out_copy(ch, q):
        return pltpu.make_async_copy(
            stags[q],
            out_hbm.at[b, :, pl.ds(n_base + ch * CH, CH)],
            osems[q])

    def prep_rid(ch, q):
        q0 = ch * KC
        for p in range(CH):
            v_a = idx_v[pl.ds(q0 + p * k, _L)] + boff
            v_b = idx_v[pl.ds(q0 + p * k + (k - _L), _L)] + boff
            rids[q][pl.ds(p * k, _L)] = v_a
            rids[q][pl.ds(p * k + (k - _L), _L)] = v_b
        p0 = boff + n_base + ch * CH
        tail = jnp.where(io < CH, io + p0, p0)
        rids[q][pl.ds(KC, _L)] = tail

    def compute(ch, q):
        rows = rowss[q]
        stag = stags[q]
        for p in range(CH):
            xr = [rows[KC + p, pl.ds(cc * _L, _L)]
                  for cc in range(C // _L)]
            for j in range(k):
                for cc in range(C // _L):
                    g = rows[p * k + j, pl.ds(cc * _L, _L)]
                    stag[j, p, pl.ds(cc * _L, _L)] = g - xr[cc]
                    stag[j, p, pl.ds(C + cc * _L, _L)] = xr[cc]

    prep_rid(0, 0)
    gather_copy(0).start()

    def pair_body(i, _):
        for s in range(2):
            ch = i * 2 + s
            @pl.when(ch + 1 < n_chunks)
            def _():
                prep_rid(ch + 1, 1 - s)
                gather_copy(1 - s).start()
            gather_copy(s).wait()
            @pl.when(i > 0)
            def _():
                out_copy(ch, s).wait()
            compute(ch, s)
            out_copy(ch, s).start()
        return ()

    lax.fori_loop(0, n_chunks // 2, pair_body, (), unroll=False)
    out_copy(n_chunks - 2, 0).wait()
    out_copy(n_chunks - 1, 1).wait()


def kernel(points, idx):
    B, N, C = points.shape
    k = idx.shape[2]
    BN = B * N
    CH = 8

    pts2 = jnp.concatenate([points, points], axis=2).reshape(BN, 2 * C)
    idx_t = idx.reshape(_NW, (BN // _NW) * k)

    mesh = plsc.VectorSubcoreMesh(core_axis_name="c", subcore_axis_name="s")
    body = functools.partial(_sc_body, CH, B, N, C, k)
    NR = k * CH + _L
    sc_fn = pl.kernel(
        body,
        out_type=jax.ShapeDtypeStruct((B, k, N, 2 * C), jnp.float32),
        mesh=mesh,
        compiler_params=pltpu.CompilerParams(needs_layout_passes=False,
                                             use_tc_tiling_on_sc=True),
        scratch_types=[
            pltpu.VMEM(((BN // _NW) * k,), jnp.int32),
            [pltpu.VMEM((NR,), jnp.int32) for _ in range(2)],
            [pltpu.VMEM((NR, 2 * C), jnp.float32) for _ in range(2)],
            [pltpu.VMEM((k, CH, 2 * C), jnp.float32) for _ in range(2)],
            [pltpu.SemaphoreType.DMA for _ in range(2)],
            [pltpu.SemaphoreType.DMA for _ in range(2)],
        ],
    )
    out = sc_fn(pts2, idx_t)
    return jnp.transpose(out, (0, 2, 3, 1))


# speedup vs baseline: 7.8110x; 1.1243x over previous
import functools

import jax
import jax.numpy as jnp
from jax import lax
from jax.experimental import pallas as pl
from jax.experimental.pallas import tpu as pltpu
from jax.experimental.pallas import tpu_sc as plsc

_NC = 2
_NS = 16
_NW = _NC * _NS
_L = 16


def _sc_body(CH, B, N, C, k, pts_hbm, idx_hbm, out_hbm,
             idx_v, rids, rowss, stags, gsems, osems):
    PPT = N * B // _NW
    KC = k * CH
    NR = KC + CH
    n_chunks = PPT // CH

    wid = lax.axis_index("s") * _NC + lax.axis_index("c")
    base_pt = wid * PPT
    b = base_pt // N
    boff = b * N
    n_base = base_pt - b * N

    io = lax.iota(jnp.int32, _L)

    pltpu.sync_copy(idx_hbm.at[wid], idx_v)

    def gather_copy(q):
        return pltpu.make_async_copy(pts_hbm.at[rids[q]],
                                     rowss[q].at[pl.ds(0, KC)], gsems[q])

    def centre_copy(ch, q):
        p0 = boff + n_base + ch * CH
        return pltpu.make_async_copy(pts_hbm.at[pl.ds(p0, CH)],
                                     rowss[q].at[pl.ds(KC, CH)], gsems[q])

    def out_copy(ch, q):
        return pltpu.make_async_copy(
            stags[q],
            out_hbm.at[b, :, pl.ds(n_base + ch * CH, CH)],
            osems[q])

    def prep_rid(ch, q):
        q0 = ch * KC
        for p in range(CH):
            v_a = idx_v[pl.ds(q0 + p * k, _L)] + boff
            v_b = idx_v[pl.ds(q0 + p * k + (k - _L), _L)] + boff
            rids[q][pl.ds(p * k, _L)] = v_a
            rids[q][pl.ds(p * k + (k - _L), _L)] = v_b

    def compute(ch, q):
        rows = rowss[q]
        stag = stags[q]
        for p in range(CH):
            xr = [rows[KC + p, pl.ds(cc * _L, _L)]
                  for cc in range(C // _L)]
            for j in range(k):
                for cc in range(C // _L):
                    g = rows[p * k + j, pl.ds(cc * _L, _L)]
                    stag[j, p, pl.ds(cc * _L, _L)] = g - xr[cc]
                    stag[j, p, pl.ds(C + cc * _L, _L)] = xr[cc]

    prep_rid(0, 0)
    gather_copy(0).start()
    centre_copy(0, 0).start()

    def pair_body(i, _):
        for s in range(2):
            ch = i * 2 + s
            @pl.when(ch + 1 < n_chunks)
            def _():
                prep_rid(ch + 1, 1 - s)
                gather_copy(1 - s).start()
                centre_copy(ch + 1, 1 - s).start()
            gather_copy(s).wait()
            centre_copy(ch, s).wait()
            @pl.when(i > 0)
            def _():
                out_copy(ch, s).wait()
            compute(ch, s)
            out_copy(ch, s).start()
        return ()

    lax.fori_loop(0, n_chunks // 2, pair_body, (), unroll=False)
    out_copy(n_chunks - 2, 0).wait()
    out_copy(n_chunks - 1, 1).wait()


def kernel(points, idx):
    B, N, C = points.shape
    k = idx.shape[2]
    BN = B * N
    CH = 8

    pts2 = jnp.concatenate([points, points], axis=2).reshape(BN, 2 * C)
    idx_t = idx.reshape(_NW, (BN // _NW) * k)

    mesh = plsc.VectorSubcoreMesh(core_axis_name="c", subcore_axis_name="s")
    body = functools.partial(_sc_body, CH, B, N, C, k)
    NR = k * CH + CH
    sc_fn = pl.kernel(
        body,
        out_type=jax.ShapeDtypeStruct((B, k, N, 2 * C), jnp.float32),
        mesh=mesh,
        compiler_params=pltpu.CompilerParams(needs_layout_passes=False,
                                             use_tc_tiling_on_sc=True),
        scratch_types=[
            pltpu.VMEM(((BN // _NW) * k,), jnp.int32),
            [pltpu.VMEM((k * CH,), jnp.int32) for _ in range(2)],
            [pltpu.VMEM((NR, 2 * C), jnp.float32) for _ in range(2)],
            [pltpu.VMEM((k, CH, 2 * C), jnp.float32) for _ in range(2)],
            [pltpu.SemaphoreType.DMA for _ in range(2)],
            [pltpu.SemaphoreType.DMA for _ in range(2)],
        ],
    )
    out = sc_fn(pts2, idx_t)
    return jnp.transpose(out, (0, 2, 3, 1))


# speedup vs baseline: 8.1020x; 1.0373x over previous
import functools

import jax
import jax.numpy as jnp
from jax import lax
from jax.experimental import pallas as pl
from jax.experimental.pallas import tpu as pltpu
from jax.experimental.pallas import tpu_sc as plsc

_NC = 2
_NS = 16
_NW = _NC * _NS
_L = 16


def _sc_body(CH, B, N, C, k, pts_hbm, idx_hbm, out_hbm,
             idx_v, rids, rowss, stags, gsems, g2sems, osems):
    PPT = N * B // _NW
    KC = k * CH
    NR = KC + CH
    n_chunks = PPT // CH

    wid = lax.axis_index("s") * _NC + lax.axis_index("c")
    base_pt = wid * PPT
    b = base_pt // N
    boff = b * N
    n_base = base_pt - b * N

    io = lax.iota(jnp.int32, _L)

    pltpu.sync_copy(idx_hbm.at[wid], idx_v)

    H = KC // 2

    def gather_copy(q):
        return pltpu.make_async_copy(pts_hbm.at[rids[q].at[pl.ds(0, H)]],
                                     rowss[q].at[pl.ds(0, H)], gsems[q])

    def gather2_copy(q):
        return pltpu.make_async_copy(pts_hbm.at[rids[q].at[pl.ds(H, H)]],
                                     rowss[q].at[pl.ds(H, H)], g2sems[q])

    def centre_copy(ch, q):
        p0 = boff + n_base + ch * CH
        return pltpu.make_async_copy(pts_hbm.at[pl.ds(p0, CH)],
                                     rowss[q].at[pl.ds(KC, CH)], gsems[q])

    def out_copy(ch, q):
        return pltpu.make_async_copy(
            stags[q],
            out_hbm.at[b, :, pl.ds(n_base + ch * CH, CH)],
            osems[q])

    def prep_rid(ch, q):
        q0 = ch * KC
        for p in range(CH):
            v_a = idx_v[pl.ds(q0 + p * k, _L)] + boff
            v_b = idx_v[pl.ds(q0 + p * k + (k - _L), _L)] + boff
            rids[q][pl.ds(p * k, _L)] = v_a
            rids[q][pl.ds(p * k + (k - _L), _L)] = v_b

    def compute(q, p_lo, p_hi):
        rows = rowss[q]
        stag = stags[q]
        for p in range(p_lo, p_hi):
            xr = [rows[KC + p, pl.ds(cc * _L, _L)]
                  for cc in range(C // _L)]
            for j in range(k):
                for cc in range(C // _L):
                    g = rows[p * k + j, pl.ds(cc * _L, _L)]
                    stag[j, p, pl.ds(cc * _L, _L)] = g - xr[cc]
                    stag[j, p, pl.ds(C + cc * _L, _L)] = xr[cc]

    prep_rid(0, 0)
    gather_copy(0).start()
    gather2_copy(0).start()
    centre_copy(0, 0).start()

    def pair_body(i, _):
        for s in range(2):
            ch = i * 2 + s
            @pl.when(ch + 1 < n_chunks)
            def _():
                prep_rid(ch + 1, 1 - s)
                gather_copy(1 - s).start()
                gather2_copy(1 - s).start()
                centre_copy(ch + 1, 1 - s).start()
            @pl.when(i > 0)
            def _():
                out_copy(ch, s).wait()
            gather_copy(s).wait()
            centre_copy(ch, s).wait()
            compute(s, 0, CH // 2)
            gather2_copy(s).wait()
            compute(s, CH // 2, CH)
            out_copy(ch, s).start()
        return ()

    lax.fori_loop(0, n_chunks // 2, pair_body, (), unroll=False)
    out_copy(n_chunks - 2, 0).wait()
    out_copy(n_chunks - 1, 1).wait()


def kernel(points, idx):
    B, N, C = points.shape
    k = idx.shape[2]
    BN = B * N
    CH = 8

    pts2 = jnp.concatenate([points, points], axis=2).reshape(BN, 2 * C)
    idx_t = idx.reshape(_NW, (BN // _NW) * k)

    mesh = plsc.VectorSubcoreMesh(core_axis_name="c", subcore_axis_name="s")
    body = functools.partial(_sc_body, CH, B, N, C, k)
    NR = k * CH + CH
    sc_fn = pl.kernel(
        body,
        out_type=jax.ShapeDtypeStruct((B, k, N, 2 * C), jnp.float32),
        mesh=mesh,
        compiler_params=pltpu.CompilerParams(needs_layout_passes=False,
                                             use_tc_tiling_on_sc=True),
        scratch_types=[
            pltpu.VMEM(((BN // _NW) * k,), jnp.int32),
            [pltpu.VMEM((k * CH,), jnp.int32) for _ in range(2)],
            [pltpu.VMEM((NR, 2 * C), jnp.float32) for _ in range(2)],
            [pltpu.VMEM((k, CH, 2 * C), jnp.float32) for _ in range(2)],
            [pltpu.SemaphoreType.DMA for _ in range(2)],
            [pltpu.SemaphoreType.DMA for _ in range(2)],
            [pltpu.SemaphoreType.DMA for _ in range(2)],
        ],
    )
    out = sc_fn(pts2, idx_t)
    return jnp.transpose(out, (0, 2, 3, 1))


# speedup vs baseline: 8.2140x; 1.0138x over previous
import functools

import jax
import jax.numpy as jnp
from jax import lax
from jax.experimental import pallas as pl
from jax.experimental.pallas import tpu as pltpu
from jax.experimental.pallas import tpu_sc as plsc

_NC = 2
_NS = 16
_NW = _NC * _NS
_L = 16


def _sc_body(CH, B, N, C, k, pts_hbm, idx_hbm, out_hbm,
             idxbs, rids, rowss, cbufs, stags, isems, gsems, osems):
    PPT = N * B // _NW
    KC = k * CH
    n_chunks = PPT // CH
    NI = 4

    wid = lax.axis_index("s") * _NC + lax.axis_index("c")
    base_pt = wid * PPT
    b = base_pt // N
    boff = b * N
    n_base = base_pt - b * N

    def idx_copy(ch, q):
        return pltpu.make_async_copy(
            idx_hbm.at[pl.ds(base_pt + ch * CH, CH)], idxbs[q], isems[q])

    def gather_copy(q):
        return pltpu.make_async_copy(pts_hbm.at[rids[q]], rowss[q], gsems[q])

    def centre_copy(ch, q):
        p0 = boff + n_base + ch * CH
        return pltpu.make_async_copy(pts_hbm.at[pl.ds(p0, CH)],
                                     cbufs[q], gsems[q])

    def out_copy(ch, q):
        return pltpu.make_async_copy(
            stags[q],
            out_hbm.at[b, :, pl.ds(n_base + ch * CH, CH)],
            osems[q])

    def prep_rid(q, iq):
        for p in range(CH):
            v_a = idxbs[iq][p, pl.ds(0, _L)] + boff
            v_b = idxbs[iq][p, pl.ds(k - _L, _L)] + boff
            rids[q][pl.ds(p * k, _L)] = v_a
            rids[q][pl.ds(p * k + (k - _L), _L)] = v_b

    def compute(q, sq):
        rows = rowss[q]
        stag = stags[sq]
        for p in range(CH):
            xr = [cbufs[q][p, pl.ds(cc * _L, _L)]
                  for cc in range(C // _L)]
            for j in range(k):
                for cc in range(C // _L):
                    g = rows[p * k + j, pl.ds(cc * _L, _L)]
                    stag[j, p, pl.ds(cc * _L, _L)] = g - xr[cc]
                    stag[j, p, pl.ds(C + cc * _L, _L)] = xr[cc]

    idx_copy(0, 0).start()
    idx_copy(1, 1).start()
    idx_copy(0, 0).wait()
    prep_rid(0, 0)
    gather_copy(0).start()
    centre_copy(0, 0).start()
    idx_copy(2, 0).start()
    idx_copy(1, 1).wait()
    prep_rid(1, 1)
    gather_copy(1).start()
    centre_copy(1, 1).start()

    def ring_body(i, _):
        for s in range(NI):
            ch = i * NI + s
            sq = s % 2
            @pl.when(ch + 3 < n_chunks)
            def _():
                idx_copy(ch + 3, (s + 1) % 2).start()
            @pl.when(ch + 2 < n_chunks)
            def _():
                q2 = (s + 2) % NI
                idx_copy(ch + 2, s % 2).wait()
                prep_rid(q2, s % 2)
                gather_copy(q2).start()
                centre_copy(ch + 2, q2).start()
            if s < 2:
                @pl.when(i > 0)
                def _():
                    out_copy(ch, sq).wait()
            else:
                out_copy(ch, sq).wait()
            gather_copy(s).wait()
            centre_copy(ch, s).wait()
            compute(s, sq)
            out_copy(ch, sq).start()
        return ()

    lax.fori_loop(0, n_chunks // NI, ring_body, (), unroll=False)
    out_copy(n_chunks - 2, 0).wait()
    out_copy(n_chunks - 1, 1).wait()


def kernel(points, idx):
    B, N, C = points.shape
    k = idx.shape[2]
    BN = B * N
    CH = 8

    pts2 = jnp.concatenate([points, points], axis=2).reshape(BN, 2 * C)
    idx2 = idx.reshape(BN, k)

    mesh = plsc.VectorSubcoreMesh(core_axis_name="c", subcore_axis_name="s")
    body = functools.partial(_sc_body, CH, B, N, C, k)
    sc_fn = pl.kernel(
        body,
        out_type=jax.ShapeDtypeStruct((B, k, N, 2 * C), jnp.float32),
        mesh=mesh,
        compiler_params=pltpu.CompilerParams(needs_layout_passes=False,
                                             use_tc_tiling_on_sc=True),
        scratch_types=[
            [pltpu.VMEM((CH, k), jnp.int32) for _ in range(2)],
            [pltpu.VMEM((k * CH,), jnp.int32) for _ in range(4)],
            [pltpu.VMEM((k * CH, 2 * C), jnp.float32) for _ in range(4)],
            [pltpu.VMEM((CH, 2 * C), jnp.float32) for _ in range(4)],
            [pltpu.VMEM((k, CH, 2 * C), jnp.float32) for _ in range(2)],
            [pltpu.SemaphoreType.DMA for _ in range(2)],
            [pltpu.SemaphoreType.DMA for _ in range(4)],
            [pltpu.SemaphoreType.DMA for _ in range(2)],
        ],
    )
    out = sc_fn(pts2, idx2)
    return jnp.transpose(out, (0, 2, 3, 1))
